# baseline (speedup 1.0000x reference)
import jax
import jax.numpy as jnp
from jax import lax
from jax.experimental import pallas as pl
from jax.experimental.pallas import tpu as pltpu
from jax.experimental.pallas import tpu_sc as plsc

_NUM_FIELDS = 26
_FIELD_SIZE = 40000
_EMB_DIM = 64
_BATCH = 4096

_NC = 2
_NS = 16
_NW = _NC * _NS
_ROWS_PER_W = _BATCH // _NW
_LANES = 16
_CHUNK = 104
_BPC = _CHUNK // _NUM_FIELDS
_NCHUNK = _ROWS_PER_W // _BPC
_IDX_PER_W = _ROWS_PER_W * _NUM_FIELDS
_DEPTH = 6
_PAIR_W = 2 * _EMB_DIM


def _body(x_hbm, table_hbm, out_hbm, idx_v, par_v, out_v, buf_v, sems):
    wid = lax.axis_index("s") * _NC + lax.axis_index("c")
    base = wid * _ROWS_PER_W

    pltpu.sync_copy(x_hbm.at[pl.ds(base * _NUM_FIELDS, _IDX_PER_W)], idx_v)

    lane_iota = lax.iota(jnp.int32, _LANES)

    def off_body(i, _):
        p0 = i * _LANES
        off = ((p0 + lane_iota) % _NUM_FIELDS) * _FIELD_SIZE
        sl = pl.ds(p0, _LANES)
        g = idx_v[sl] + off
        idx_v[sl] = lax.shift_right_logical(g, 1)
        par_v[sl] = lax.bitwise_and(g, 1)
        return 0

    lax.fori_loop(0, _IDX_PER_W // _LANES, off_body, 0, unroll=8)

    def start(k, par):
        pltpu.async_copy(
            table_hbm.at[idx_v.at[pl.ds(k * _CHUNK, _CHUNK)]],
            buf_v.at[par],
            sems.at[par],
        )

    def wait_chunk(par):
        pltpu.make_async_copy(
            table_hbm.at[pl.ds(0, _CHUNK), :], buf_v.at[par], sems.at[par]
        ).wait()

    scale = jnp.float32(1.0 / _NUM_FIELDS)

    def process(k, par):
        pvecs = [
            par_v[pl.ds(k * _CHUNK + g * _LANES, _LANES)]
            for g in range(_CHUNK // _LANES)
        ]
        pvec_tail = par_v[pl.ds(k * _CHUNK + _CHUNK - _LANES, _LANES)]

        def parity(p):
            if p < (_CHUNK // _LANES) * _LANES:
                return pvecs[p // _LANES][p % _LANES]
            return pvec_tail[p - (_CHUNK - _LANES)]

        for t in range(_BPC):
            row0 = t * _NUM_FIELDS
            accs = None
            for f in range(_NUM_FIELDS):
                p = row0 + f
                half = parity(p) * _EMB_DIM
                vals = [
                    buf_v[par, p, pl.ds(half + j * _LANES, _LANES)]
                    for j in range(_EMB_DIM // _LANES)
                ]
                if accs is None:
                    accs = vals
                else:
                    accs = [a + v for a, v in zip(accs, vals)]
            ob = (t % 2) * _EMB_DIM
            for j in range(_EMB_DIM // _LANES):
                out_v[k * (_BPC // 2) + t // 2, pl.ds(ob + j * _LANES, _LANES)] = (
                    accs[j] * scale
                )

    @pl.loop(0, _DEPTH)
    def prime_loop(k):
        start(k, k)

    @pl.loop(0, _NCHUNK)
    def chunk_loop(k):
        par = lax.rem(k, _DEPTH)
        wait_chunk(par)
        process(k, par)

        @pl.when(k + _DEPTH < _NCHUNK)
        def _():
            start(k + _DEPTH, par)

    pltpu.sync_copy(out_v, out_hbm.at[pl.ds(wid * (_ROWS_PER_W // 2), _ROWS_PER_W // 2), :])


@jax.jit
def kernel(x, table):
    run = pl.kernel(
        _body,
        out_type=jax.ShapeDtypeStruct((_BATCH // 2, _PAIR_W), jnp.float32),
        mesh=plsc.VectorSubcoreMesh(core_axis_name="c", subcore_axis_name="s"),
        scratch_types=[
            pltpu.VMEM((_IDX_PER_W,), jnp.int32),
            pltpu.VMEM((_IDX_PER_W,), jnp.int32),
            pltpu.VMEM((_ROWS_PER_W // 2, _PAIR_W), jnp.float32),
            pltpu.VMEM((_DEPTH, _CHUNK, _PAIR_W), jnp.float32),
            pltpu.SemaphoreType.DMA((_DEPTH,)),
        ],
    )
    out2 = run(x.reshape(-1), table.reshape(-1, _PAIR_W))
    return out2.reshape(_BATCH, _EMB_DIM)
